# baseline (speedup 1.0000x reference)
import functools

import jax
import jax.numpy as jnp
from jax import lax
from jax.experimental import pallas as pl
from jax.experimental.pallas import tpu as pltpu
from jax.experimental.pallas import tpu_sc as plsc

VOCAB = 1000000
DIM = 16
BATCH = 16384
WIDTH = 26

NC = 2
NS = 16
NW = NC * NS
LANES = 16

B_FLAT = BATCH * WIDTH
B_PER_W = B_FLAT // NW
IDX_COLS = 128
IDX_ROWS = B_PER_W // IDX_COLS
CHUNK_IDX_ROWS = 26
CHUNK_ROWS = CHUNK_IDX_ROWS * IDX_COLS
NUM_CHUNKS = IDX_ROWS // CHUNK_IDX_ROWS
GROUPS_PER_CHUNK = CHUNK_ROWS // LANES


def _body(idx_hbm, table_hbm, out_hbm, idx_v, rows_v, sem):
    wid = lax.axis_index("s") * NC + lax.axis_index("c")
    base = wid * B_PER_W
    pltpu.sync_copy(idx_hbm.at[wid], idx_v)

    for c in range(NUM_CHUNKS):
        copies = [
            pltpu.async_copy(
                table_hbm.at[idx_v.at[c * CHUNK_IDX_ROWS + j]],
                rows_v.at[pl.ds(j * IDX_COLS, IDX_COLS)],
                sem,
            )
            for j in range(CHUNK_IDX_ROWS)
        ]
        for cp in copies:
            cp.wait()

        def mask_group(g, _, c=c):
            row = c * CHUNK_IDX_ROWS + g // 8
            col = (g % 8) * LANES
            iv = idx_v[row, pl.ds(col, LANES)]
            m = iv == 0
            rowids = lax.iota(jnp.int32, LANES) + g * LANES
            zeros = jnp.zeros((LANES,), jnp.float32)
            for colid in range(DIM):
                plsc.store_scatter(
                    rows_v,
                    [rowids, jnp.full((LANES,), colid, jnp.int32)],
                    zeros,
                    mask=m,
                )
            return 0

        lax.fori_loop(0, GROUPS_PER_CHUNK, mask_group, 0)
        pltpu.sync_copy(
            rows_v, out_hbm.at[pl.ds(base + c * CHUNK_ROWS, CHUNK_ROWS)]
        )


@jax.jit
def _embed(idx, table):
    mesh = plsc.VectorSubcoreMesh(core_axis_name="c", subcore_axis_name="s")
    kern = functools.partial(
        pl.kernel,
        out_type=jax.ShapeDtypeStruct((B_FLAT, DIM), jnp.float32),
        mesh=mesh,
        scratch_types=[
            pltpu.VMEM((IDX_ROWS, IDX_COLS), jnp.int32),
            pltpu.VMEM((CHUNK_ROWS, DIM), jnp.float32),
            pltpu.SemaphoreType.DMA,
        ],
        compiler_params=pltpu.CompilerParams(
            needs_layout_passes=False, use_tc_tiling_on_sc=False
        ),
    )(_body)
    return kern(idx, table)


def kernel(input, table):
    mask = (input != 0).astype(jnp.float32)[..., None]
    idx = jnp.maximum(input, 0).astype(jnp.int32).reshape(NW, IDX_ROWS, IDX_COLS)
    tbl = jnp.maximum(table, jnp.float32(-3.0e38))
    out = _embed(idx, tbl)
    return out.reshape(BATCH, WIDTH, DIM) * mask


# speedup vs baseline: 1.7326x; 1.7326x over previous
import functools

import jax
import jax.numpy as jnp
from jax import lax
from jax.experimental import pallas as pl
from jax.experimental.pallas import tpu as pltpu
from jax.experimental.pallas import tpu_sc as plsc

VOCAB = 1000000
DIM = 16
BATCH = 16384
WIDTH = 26

NC = 2
NS = 16
NW = NC * NS
LANES = 16

B_FLAT = BATCH * WIDTH
B_PER_W = B_FLAT // NW
IDX_COLS = 128
IDX_ROWS = B_PER_W // IDX_COLS
CHUNK_IDX_ROWS = 26
CHUNK_ROWS = CHUNK_IDX_ROWS * IDX_COLS
NUM_CHUNKS = IDX_ROWS // CHUNK_IDX_ROWS
GROUPS_PER_CHUNK = CHUNK_ROWS // LANES


def _body(idx_hbm, table_hbm, out_hbm, idx_v, rows_v, planes_v, sem):
    wid = lax.axis_index("s") * NC + lax.axis_index("c")
    base = wid * B_PER_W
    pltpu.sync_copy(idx_hbm.at[wid], idx_v)

    for c in range(NUM_CHUNKS):
        copies = [
            pltpu.async_copy(
                table_hbm.at[idx_v.at[c * CHUNK_IDX_ROWS + j]],
                rows_v.at[pl.ds(j * IDX_COLS, IDX_COLS)],
                sem,
            )
            for j in range(CHUNK_IDX_ROWS)
        ]
        for cp in copies:
            cp.wait()

        def transpose_group(g, _, c=c):
            row = c * CHUNK_IDX_ROWS + g // 8
            col = (g % 8) * LANES
            iv = idx_v[row, pl.ds(col, LANES)]
            m = iv == 0
            zeros = jnp.zeros((LANES,), jnp.float32)
            rowids = lax.iota(jnp.int32, LANES) + g * LANES
            for d in range(DIM):
                v = plsc.load_gather(
                    rows_v, [rowids, jnp.full((LANES,), d, jnp.int32)]
                )
                planes_v[d, pl.ds(g * LANES, LANES)] = jnp.where(m, zeros, v)
            return 0

        lax.fori_loop(0, GROUPS_PER_CHUNK, transpose_group, 0)

        out_copies = [
            pltpu.async_copy(
                planes_v.at[d],
                out_hbm.at[d, pl.ds(base + c * CHUNK_ROWS, CHUNK_ROWS)],
                sem,
            )
            for d in range(DIM)
        ]
        for cp in out_copies:
            cp.wait()


@jax.jit
def _embed(idx, table):
    mesh = plsc.VectorSubcoreMesh(core_axis_name="c", subcore_axis_name="s")
    kern = functools.partial(
        pl.kernel,
        out_type=jax.ShapeDtypeStruct((DIM, B_FLAT), jnp.float32),
        mesh=mesh,
        scratch_types=[
            pltpu.VMEM((IDX_ROWS, IDX_COLS), jnp.int32),
            pltpu.VMEM((CHUNK_ROWS, DIM), jnp.float32),
            pltpu.VMEM((DIM, CHUNK_ROWS), jnp.float32),
            pltpu.SemaphoreType.DMA,
        ],
        compiler_params=pltpu.CompilerParams(
            needs_layout_passes=False, use_tc_tiling_on_sc=False
        ),
    )(_body)
    return kern(idx, table)


def kernel(input, table):
    idx = input.astype(jnp.int32).reshape(NW, IDX_ROWS, IDX_COLS)
    out_t = _embed(idx, table)
    return out_t.reshape(DIM, BATCH, WIDTH).transpose(1, 2, 0)


# speedup vs baseline: 1.7334x; 1.0005x over previous
import functools

import jax
import jax.numpy as jnp
from jax import lax
from jax.experimental import pallas as pl
from jax.experimental.pallas import tpu as pltpu
from jax.experimental.pallas import tpu_sc as plsc

VOCAB = 1000000
DIM = 16
BATCH = 16384
WIDTH = 26

NC = 2
NS = 16
NW = NC * NS
LANES = 16

B_FLAT = BATCH * WIDTH
B_PER_W = B_FLAT // NW
IDX_COLS = 128
IDX_ROWS = B_PER_W // IDX_COLS
CHUNK_IDX_ROWS = 26
CHUNK_ROWS = CHUNK_IDX_ROWS * IDX_COLS
NUM_CHUNKS = IDX_ROWS // CHUNK_IDX_ROWS
GROUPS_PER_CHUNK = CHUNK_ROWS // LANES


def _body(idx_hbm, table_hbm, out_hbm, idx_v, rows_v, planes_v, sem):
    wid = lax.axis_index("s") * NC + lax.axis_index("c")
    base = wid * B_PER_W
    pltpu.sync_copy(idx_hbm.at[wid], idx_v)

    for c in range(NUM_CHUNKS):
        copies = [
            pltpu.async_copy(
                table_hbm.at[idx_v.at[c * CHUNK_IDX_ROWS + j]],
                rows_v.at[pl.ds(j * IDX_COLS, IDX_COLS)],
                sem,
            )
            for j in range(CHUNK_IDX_ROWS)
        ]
        for cp in copies:
            cp.wait()

        def transpose_group(g, _, c=c):
            row = c * CHUNK_IDX_ROWS + g // 8
            col = (g % 8) * LANES
            iv = idx_v[row, pl.ds(col, LANES)]
            m = iv == 0
            zeros = jnp.zeros((LANES,), jnp.float32)
            rowids = lax.iota(jnp.int32, LANES) + g * LANES
            for d in range(DIM):
                v = plsc.load_gather(
                    rows_v, [rowids, jnp.full((LANES,), d, jnp.int32)]
                )
                planes_v[d, pl.ds(g * LANES, LANES)] = jnp.where(m, zeros, v)
            return 0

        lax.fori_loop(0, GROUPS_PER_CHUNK, transpose_group, 0)

        out_copies = [
            pltpu.async_copy(
                planes_v.at[d],
                out_hbm.at[d, pl.ds(base + c * CHUNK_ROWS, CHUNK_ROWS)],
                sem,
            )
            for d in range(DIM)
        ]
        for cp in out_copies:
            cp.wait()


@jax.jit
def _embed(idx, table):
    mesh = plsc.VectorSubcoreMesh(core_axis_name="c", subcore_axis_name="s")
    kern = functools.partial(
        pl.kernel,
        out_type=jax.ShapeDtypeStruct((DIM, B_FLAT), jnp.float32),
        mesh=mesh,
        scratch_types=[
            pltpu.VMEM((IDX_ROWS, IDX_COLS), jnp.int32),
            pltpu.VMEM((CHUNK_ROWS, DIM), jnp.float32),
            pltpu.VMEM((DIM, CHUNK_ROWS), jnp.float32),
            pltpu.SemaphoreType.DMA,
        ],
        compiler_params=pltpu.CompilerParams(
            needs_layout_passes=False, use_tc_tiling_on_sc=False
        ),
    )(_body)
    return kern(idx, table)


def kernel(input, table):
    idx = input.astype(jnp.int32).reshape(NW, IDX_ROWS, IDX_COLS)
    tbl = jax.lax.optimization_barrier(table.reshape(VOCAB // 8, DIM * 8))
    tbl = tbl.reshape(VOCAB, DIM)
    out_t = _embed(idx, tbl)
    return out_t.reshape(DIM, BATCH, WIDTH).transpose(1, 2, 0)


# speedup vs baseline: 1.7808x; 1.0273x over previous
import functools

import jax
import jax.numpy as jnp
from jax import lax
from jax.experimental import pallas as pl
from jax.experimental.pallas import tpu as pltpu
from jax.experimental.pallas import tpu_sc as plsc

VOCAB = 1000000
DIM = 16
BATCH = 16384
WIDTH = 26

NC = 2
NS = 16
NW = NC * NS
LANES = 16

B_FLAT = BATCH * WIDTH
B_PER_W = B_FLAT // NW
IDX_COLS = 128
IDX_ROWS = B_PER_W // IDX_COLS
CHUNK_IDX_ROWS = 13
CHUNK_ROWS = CHUNK_IDX_ROWS * IDX_COLS
NUM_CHUNKS = IDX_ROWS // CHUNK_IDX_ROWS
GROUPS_PER_CHUNK = CHUNK_ROWS // LANES
NBUF = 2


def _fire_gathers(table_hbm, idx_v, rows_v, sem, c):
    return [
        pltpu.async_copy(
            table_hbm.at[idx_v.at[c * CHUNK_IDX_ROWS + j]],
            rows_v.at[pl.ds(j * IDX_COLS, IDX_COLS)],
            sem,
        )
        for j in range(CHUNK_IDX_ROWS)
    ]


def _body(idx_hbm, table_hbm, out_hbm, idx_v, rows_v, planes_v, sem, osem):
    wid = lax.axis_index("s") * NC + lax.axis_index("c")
    base = wid * B_PER_W
    pltpu.sync_copy(idx_hbm.at[wid], idx_v)

    gathers = {0: _fire_gathers(table_hbm, idx_v, rows_v.at[0], sem, 0)}
    out_copies = {}
    for c in range(NUM_CHUNKS):
        buf = c % NBUF
        if c + 1 < NUM_CHUNKS:
            gathers[c + 1] = _fire_gathers(
                table_hbm, idx_v, rows_v.at[(c + 1) % NBUF], sem, c + 1
            )
        for cp in gathers.pop(c):
            cp.wait()
        if c - NBUF in out_copies:
            for cp in out_copies.pop(c - NBUF):
                cp.wait()

        def transpose_group(g, _, c=c, buf=buf):
            row = c * CHUNK_IDX_ROWS + g // 8
            col = (g % 8) * LANES
            iv = idx_v[row, pl.ds(col, LANES)]
            m = iv == 0
            zeros = jnp.zeros((LANES,), jnp.float32)
            rowids = lax.iota(jnp.int32, LANES) + g * LANES
            for d in range(DIM):
                v = plsc.load_gather(
                    rows_v.at[buf], [rowids, jnp.full((LANES,), d, jnp.int32)]
                )
                planes_v[buf, d, pl.ds(g * LANES, LANES)] = jnp.where(m, zeros, v)
            return 0

        lax.fori_loop(0, GROUPS_PER_CHUNK, transpose_group, 0)

        out_copies[c] = [
            pltpu.async_copy(
                planes_v.at[buf, d],
                out_hbm.at[d, pl.ds(base + c * CHUNK_ROWS, CHUNK_ROWS)],
                osem,
            )
            for d in range(DIM)
        ]
    for c in list(out_copies):
        for cp in out_copies.pop(c):
            cp.wait()


@jax.jit
def _embed(idx, table):
    mesh = plsc.VectorSubcoreMesh(core_axis_name="c", subcore_axis_name="s")
    kern = functools.partial(
        pl.kernel,
        out_type=jax.ShapeDtypeStruct((DIM, B_FLAT), jnp.float32),
        mesh=mesh,
        scratch_types=[
            pltpu.VMEM((IDX_ROWS, IDX_COLS), jnp.int32),
            pltpu.VMEM((NBUF, CHUNK_ROWS, DIM), jnp.float32),
            pltpu.VMEM((NBUF, DIM, CHUNK_ROWS), jnp.float32),
            pltpu.SemaphoreType.DMA,
            pltpu.SemaphoreType.DMA,
        ],
        compiler_params=pltpu.CompilerParams(
            needs_layout_passes=False, use_tc_tiling_on_sc=False
        ),
    )(_body)
    return kern(idx, table)


def kernel(input, table):
    idx = input.astype(jnp.int32).reshape(NW, IDX_ROWS, IDX_COLS)
    out_t = _embed(idx, table)
    return out_t.reshape(DIM, BATCH, WIDTH).transpose(1, 2, 0)
